# baseline (speedup 1.0000x reference)
import jax
import jax.numpy as jnp
from jax.experimental import pallas as pl
from jax.experimental.pallas import tpu as pltpu

_SCALES = (1.0, 2.0, 4.0, 8.0)
_N_CODES = 1024
_CODE_DIM = 64
_HIDDEN = 512
_T_LEN = 2048
_BATCH = 16
_N_Q = 4
_TBLK = 512


def _rvq_kernel(x_ref, cb_ref, win_ref, wout_ref, zq_ref, codes_ref, loss_ref):
    f32 = jnp.float32
    r = x_ref[0]
    tblk = r.shape[1]
    zq_acc = jnp.zeros_like(r)
    loss_vec = jnp.zeros((tblk, 1), dtype=f32)
    iota = jax.lax.broadcasted_iota(jnp.int32, (tblk, _N_CODES), 1)
    code_rows = []
    for i, s in enumerate(_SCALES):
        wi = win_ref[i]
        wo = wout_ref[i]
        cb = cb_ref[i]
        z_e = jax.lax.dot_general(
            wi, r, (((1,), (0,)), ((), ())), preferred_element_type=f32
        ) * (1.0 / s)
        ze_t = z_e.T
        scores = jax.lax.dot_general(
            ze_t, cb.T, (((1,), (0,)), ((), ())), preferred_element_type=f32
        )
        cb_sq = jnp.sum(cb * cb, axis=1)[None, :]
        ze_sq = jnp.sum(ze_t * ze_t, axis=1, keepdims=True)
        dist = ze_sq - 2.0 * scores + cb_sq
        dmin = jnp.min(dist, axis=1, keepdims=True)
        idx = jnp.min(
            jnp.where(dist == dmin, iota, _N_CODES), axis=1, keepdims=True
        )
        onehot = (iota == idx).astype(f32)
        bf16 = jnp.bfloat16
        c1 = cb.astype(bf16)
        t1 = cb - c1.astype(f32)
        c2 = t1.astype(bf16)
        c3 = (t1 - c2.astype(f32)).astype(bf16)
        onehot_b = onehot.astype(bf16)
        zq_t = (
            jax.lax.dot_general(
                onehot_b, c1, (((1,), (0,)), ((), ())), preferred_element_type=f32
            )
            + jax.lax.dot_general(
                onehot_b, c2, (((1,), (0,)), ((), ())), preferred_element_type=f32
            )
        ) + jax.lax.dot_general(
            onehot_b, c3, (((1,), (0,)), ((), ())), preferred_element_type=f32
        )
        d = ze_t - zq_t
        loss_vec = loss_vec + jnp.sum(d * d, axis=1, keepdims=True)
        zq_st_t = ze_t + (zq_t - ze_t)
        zq_i = (jax.lax.dot_general(
            zq_st_t, wo, (((1,), (1,)), ((), ())), preferred_element_type=f32
        ) * s).T
        r = r - zq_i
        zq_acc = zq_acc + zq_i
        code_rows.append(idx.T)
    zq_ref[0] = zq_acc
    codes_ref[0] = jnp.concatenate(code_rows, axis=0)
    loss_ref[0] = jnp.full((8, 128), jnp.sum(loss_vec), dtype=f32)


def kernel(xin, codebooks, W_in, W_out):
    b, d, t = xin.shape
    n_tb = t // _TBLK
    grid = (b, n_tb)
    out_shape = [
        jax.ShapeDtypeStruct((b, d, t), jnp.float32),
        jax.ShapeDtypeStruct((b, _N_Q, t), jnp.int32),
        jax.ShapeDtypeStruct((b, 8 * n_tb, 128), jnp.float32),
    ]
    in_specs = [
        pl.BlockSpec((1, d, _TBLK), lambda i, j: (i, 0, j)),
        pl.BlockSpec((_N_Q, _N_CODES, _CODE_DIM), lambda i, j: (0, 0, 0)),
        pl.BlockSpec((_N_Q, _CODE_DIM, d), lambda i, j: (0, 0, 0)),
        pl.BlockSpec((_N_Q, d, _CODE_DIM), lambda i, j: (0, 0, 0)),
    ]
    out_specs = [
        pl.BlockSpec((1, d, _TBLK), lambda i, j: (i, 0, j)),
        pl.BlockSpec((1, _N_Q, _TBLK), lambda i, j: (i, 0, j)),
        pl.BlockSpec((1, 8, 128), lambda i, j: (i, j, 0)),
    ]
    z_q, codes_bqt, loss_parts = pl.pallas_call(
        _rvq_kernel,
        grid=grid,
        in_specs=in_specs,
        out_specs=out_specs,
        out_shape=out_shape,
        compiler_params=pltpu.CompilerParams(
            dimension_semantics=("parallel", "parallel"),
        ),
    )(xin, codebooks, W_in, W_out)
    codes = codes_bqt.transpose(1, 0, 2)
    total_sq = jnp.sum(loss_parts) / 1024.0
    loss = total_sq * (1.25 / (b * t * _CODE_DIM))
    return z_q, loss, codes


# speedup vs baseline: 1.5921x; 1.5921x over previous
import jax
import jax.numpy as jnp
from jax.experimental import pallas as pl
from jax.experimental.pallas import tpu as pltpu

_SCALES = (1.0, 2.0, 4.0, 8.0)
_N_CODES = 1024
_CODE_DIM = 64
_HIDDEN = 512
_T_LEN = 2048
_BATCH = 16
_N_Q = 4
_TBLK = 512


def _rvq_kernel(x_ref, cb_ref, win_ref, wout_ref, zq_ref, codes_ref, loss_ref):
    f32 = jnp.float32
    r = x_ref[0]
    tblk = r.shape[1]
    zq_acc = jnp.zeros_like(r)
    loss_vec = jnp.zeros((1, tblk), dtype=f32)
    iota = jax.lax.broadcasted_iota(jnp.int32, (_N_CODES, tblk), 0)
    code_rows = []
    for i, s in enumerate(_SCALES):
        wi = win_ref[i]
        wo = wout_ref[i]
        cb = cb_ref[i]
        z_e = jax.lax.dot_general(
            wi, r, (((1,), (0,)), ((), ())), preferred_element_type=f32
        ) * (1.0 / s)
        scores = jax.lax.dot_general(
            cb, z_e, (((1,), (0,)), ((), ())), preferred_element_type=f32
        )
        cb_sq = jnp.sum(cb * cb, axis=1, keepdims=True)
        ze_sq = jnp.sum(z_e * z_e, axis=0, keepdims=True)
        dist = ze_sq - 2.0 * scores + cb_sq
        dmin = jnp.min(dist, axis=0, keepdims=True)
        idx = jnp.min(
            jnp.where(dist == dmin, iota, _N_CODES), axis=0, keepdims=True
        )
        onehot = (iota == idx).astype(jnp.bfloat16)
        bf16 = jnp.bfloat16
        c1 = cb.astype(bf16)
        t1 = cb - c1.astype(f32)
        c2 = t1.astype(bf16)
        c3 = (t1 - c2.astype(f32)).astype(bf16)
        zq = (
            jax.lax.dot_general(
                c1, onehot, (((0,), (0,)), ((), ())), preferred_element_type=f32
            )
            + jax.lax.dot_general(
                c2, onehot, (((0,), (0,)), ((), ())), preferred_element_type=f32
            )
        ) + jax.lax.dot_general(
            c3, onehot, (((0,), (0,)), ((), ())), preferred_element_type=f32
        )
        d = z_e - zq
        loss_vec = loss_vec + jnp.sum(d * d, axis=0, keepdims=True)
        zq_st = z_e + (zq - z_e)
        zq_i = jax.lax.dot_general(
            wo, zq_st, (((1,), (0,)), ((), ())), preferred_element_type=f32
        ) * s
        r = r - zq_i
        zq_acc = zq_acc + zq_i
        code_rows.append(idx)
    zq_ref[0] = zq_acc
    codes_ref[0] = jnp.concatenate(code_rows, axis=0)
    loss_ref[0] = jnp.full((8, 128), jnp.sum(loss_vec), dtype=f32)


def kernel(xin, codebooks, W_in, W_out):
    b, d, t = xin.shape
    n_tb = t // _TBLK
    grid = (b, n_tb)
    out_shape = [
        jax.ShapeDtypeStruct((b, d, t), jnp.float32),
        jax.ShapeDtypeStruct((b, _N_Q, t), jnp.int32),
        jax.ShapeDtypeStruct((b, 8 * n_tb, 128), jnp.float32),
    ]
    in_specs = [
        pl.BlockSpec((1, d, _TBLK), lambda i, j: (i, 0, j)),
        pl.BlockSpec((_N_Q, _N_CODES, _CODE_DIM), lambda i, j: (0, 0, 0)),
        pl.BlockSpec((_N_Q, _CODE_DIM, d), lambda i, j: (0, 0, 0)),
        pl.BlockSpec((_N_Q, d, _CODE_DIM), lambda i, j: (0, 0, 0)),
    ]
    out_specs = [
        pl.BlockSpec((1, d, _TBLK), lambda i, j: (i, 0, j)),
        pl.BlockSpec((1, _N_Q, _TBLK), lambda i, j: (i, 0, j)),
        pl.BlockSpec((1, 8, 128), lambda i, j: (i, j, 0)),
    ]
    z_q, codes_bqt, loss_parts = pl.pallas_call(
        _rvq_kernel,
        grid=grid,
        in_specs=in_specs,
        out_specs=out_specs,
        out_shape=out_shape,
        compiler_params=pltpu.CompilerParams(
            dimension_semantics=("parallel", "parallel"),
        ),
    )(xin, codebooks, W_in, W_out)
    codes = codes_bqt.transpose(1, 0, 2)
    total_sq = jnp.sum(loss_parts) / 1024.0
    loss = total_sq * (1.25 / (b * t * _CODE_DIM))
    return z_q, loss, codes


# speedup vs baseline: 1.7130x; 1.0759x over previous
import jax
import jax.numpy as jnp
from jax.experimental import pallas as pl
from jax.experimental.pallas import tpu as pltpu

_SCALES = (1.0, 2.0, 4.0, 8.0)
_N_CODES = 1024
_CODE_DIM = 64
_N_Q = 4
_TBLK = 512


def _rvq_kernel(x_ref, cb_ref, win_ref, wout_ref,
                zq_ref, codes_ref, loss_ref):
    f32 = jnp.float32
    r = x_ref[0]
    tblk = r.shape[1]
    zq_acc = jnp.zeros_like(r)
    loss_vec = jnp.zeros((1, tblk), dtype=f32)
    iota = jax.lax.broadcasted_iota(jnp.int32, (_N_CODES, tblk), 0)
    code_rows = []
    for i, s in enumerate(_SCALES):
        wi = win_ref[i]
        wo = wout_ref[i]
        cb = cb_ref[i]
        z_e = jax.lax.dot_general(
            wi, r, (((1,), (0,)), ((), ())), preferred_element_type=f32
        ) * (1.0 / s)
        scores = jax.lax.dot_general(
            cb, z_e, (((1,), (0,)), ((), ())), preferred_element_type=f32
        )
        cb_sq = jnp.sum(cb * cb, axis=1, keepdims=True)
        ze_sq = jnp.sum(z_e * z_e, axis=0, keepdims=True)
        dist = ze_sq - 2.0 * scores + cb_sq
        dmin = jnp.min(dist, axis=0, keepdims=True)
        idx = jnp.min(
            jnp.where(dist == dmin, iota, _N_CODES), axis=0, keepdims=True
        )
        onehot = (iota == idx).astype(jnp.bfloat16)
        bf16 = jnp.bfloat16
        c1 = cb.astype(bf16)
        t1 = cb - c1.astype(f32)
        c2 = t1.astype(bf16)
        c3 = (t1 - c2.astype(f32)).astype(bf16)
        cpk = jnp.concatenate([c1, c2, c3], axis=1)
        zq3 = jax.lax.dot_general(
            cpk, onehot, (((0,), (0,)), ((), ())), preferred_element_type=f32
        )
        zq = (zq3[0:_CODE_DIM] + zq3[_CODE_DIM:2 * _CODE_DIM]) \
            + zq3[2 * _CODE_DIM:3 * _CODE_DIM]
        d = z_e - zq
        loss_vec = loss_vec + jnp.sum(d * d, axis=0, keepdims=True)
        zq_st = z_e + (zq - z_e)
        zq_i = jax.lax.dot_general(
            wo, zq_st, (((1,), (0,)), ((), ())), preferred_element_type=f32
        ) * s
        r = r - zq_i
        zq_acc = zq_acc + zq_i
        code_rows.append(idx)
    zq_ref[0] = zq_acc
    codes_ref[0] = jnp.concatenate(code_rows, axis=0)
    loss_ref[0] = jnp.full((8, 128), jnp.sum(loss_vec), dtype=f32)


def kernel(xin, codebooks, W_in, W_out):
    b, d, t = xin.shape
    n_tb = t // _TBLK
    f32 = jnp.float32
    grid = (b, n_tb)
    out_shape = [
        jax.ShapeDtypeStruct((b, d, t), f32),
        jax.ShapeDtypeStruct((b, _N_Q, t), jnp.int32),
        jax.ShapeDtypeStruct((b, 8 * n_tb, 128), f32),
    ]
    in_specs = [
        pl.BlockSpec((1, d, _TBLK), lambda i, j: (i, 0, j)),
        pl.BlockSpec((_N_Q, _N_CODES, _CODE_DIM), lambda i, j: (0, 0, 0)),
        pl.BlockSpec((_N_Q, _CODE_DIM, d), lambda i, j: (0, 0, 0)),
        pl.BlockSpec((_N_Q, d, _CODE_DIM), lambda i, j: (0, 0, 0)),
    ]
    out_specs = [
        pl.BlockSpec((1, d, _TBLK), lambda i, j: (i, 0, j)),
        pl.BlockSpec((1, _N_Q, _TBLK), lambda i, j: (i, 0, j)),
        pl.BlockSpec((1, 8, 128), lambda i, j: (i, j, 0)),
    ]
    z_q, codes_bqt, loss_parts = pl.pallas_call(
        _rvq_kernel,
        grid=grid,
        in_specs=in_specs,
        out_specs=out_specs,
        out_shape=out_shape,
        compiler_params=pltpu.CompilerParams(
            dimension_semantics=("parallel", "parallel"),
        ),
    )(xin, codebooks, W_in, W_out)
    codes = codes_bqt.transpose(1, 0, 2)
    total_sq = jnp.sum(loss_parts) / 1024.0
    loss = total_sq * (1.25 / (b * t * _CODE_DIM))
    return z_q, loss, codes


# speedup vs baseline: 2.4988x; 1.4588x over previous
import jax
import jax.numpy as jnp
from jax.experimental import pallas as pl
from jax.experimental.pallas import tpu as pltpu

_SCALES = (1.0, 2.0, 4.0, 8.0)
_N_CODES = 1024
_CODE_DIM = 64
_N_Q = 4
_TBLK = 1024


def _rvq_kernel(x_ref, cb_ref, win_ref, wout_ref,
                zq_ref, codes_ref, loss_ref):
    f32 = jnp.float32
    r = x_ref[0]
    tblk = r.shape[1]
    zq_acc = jnp.zeros_like(r)
    loss_vec = jnp.zeros((1, tblk), dtype=f32)
    iota = jax.lax.broadcasted_iota(jnp.int32, (_N_CODES, tblk), 0)
    code_rows = []
    for i, s in enumerate(_SCALES):
        wi = win_ref[i]
        wo = wout_ref[i]
        cb = cb_ref[i]
        z_e = jax.lax.dot_general(
            wi, r, (((1,), (0,)), ((), ())), preferred_element_type=f32
        ) * (1.0 / s)
        scores = jax.lax.dot_general(
            cb, z_e, (((1,), (0,)), ((), ())), preferred_element_type=f32
        )
        cb_sq = jnp.sum(cb * cb, axis=1, keepdims=True)
        ze_sq = jnp.sum(z_e * z_e, axis=0, keepdims=True)
        dist = ze_sq - 2.0 * scores + cb_sq
        dmin = jnp.min(dist, axis=0, keepdims=True)
        idx = jnp.min(
            jnp.where(dist == dmin, iota, _N_CODES), axis=0, keepdims=True
        )
        onehot = (iota == idx).astype(jnp.bfloat16)
        bf16 = jnp.bfloat16
        c1 = cb.astype(bf16)
        t1 = cb - c1.astype(f32)
        c2 = t1.astype(bf16)
        c3 = (t1 - c2.astype(f32)).astype(bf16)
        cpk = jnp.concatenate([c1, c2, c3], axis=1)
        zq3 = jax.lax.dot_general(
            cpk, onehot, (((0,), (0,)), ((), ())), preferred_element_type=f32
        )
        zq = (zq3[0:_CODE_DIM] + zq3[_CODE_DIM:2 * _CODE_DIM]) \
            + zq3[2 * _CODE_DIM:3 * _CODE_DIM]
        d = z_e - zq
        loss_vec = loss_vec + jnp.sum(d * d, axis=0, keepdims=True)
        zq_st = z_e + (zq - z_e)
        zq_i = jax.lax.dot_general(
            wo, zq_st, (((1,), (0,)), ((), ())), preferred_element_type=f32
        ) * s
        r = r - zq_i
        zq_acc = zq_acc + zq_i
        code_rows.append(idx)
    zq_ref[0] = zq_acc
    codes_ref[0] = jnp.concatenate(code_rows, axis=0)
    loss_ref[0] = jnp.full((8, 128), jnp.sum(loss_vec), dtype=f32)


def kernel(xin, codebooks, W_in, W_out):
    b, d, t = xin.shape
    n_tb = t // _TBLK
    f32 = jnp.float32
    grid = (b, n_tb)
    out_shape = [
        jax.ShapeDtypeStruct((b, d, t), f32),
        jax.ShapeDtypeStruct((b, _N_Q, t), jnp.int32),
        jax.ShapeDtypeStruct((b, 8 * n_tb, 128), f32),
    ]
    in_specs = [
        pl.BlockSpec((1, d, _TBLK), lambda i, j: (i, 0, j)),
        pl.BlockSpec((_N_Q, _N_CODES, _CODE_DIM), lambda i, j: (0, 0, 0)),
        pl.BlockSpec((_N_Q, _CODE_DIM, d), lambda i, j: (0, 0, 0)),
        pl.BlockSpec((_N_Q, d, _CODE_DIM), lambda i, j: (0, 0, 0)),
    ]
    out_specs = [
        pl.BlockSpec((1, d, _TBLK), lambda i, j: (i, 0, j)),
        pl.BlockSpec((1, _N_Q, _TBLK), lambda i, j: (i, 0, j)),
        pl.BlockSpec((1, 8, 128), lambda i, j: (i, j, 0)),
    ]
    z_q, codes_bqt, loss_parts = pl.pallas_call(
        _rvq_kernel,
        grid=grid,
        in_specs=in_specs,
        out_specs=out_specs,
        out_shape=out_shape,
        compiler_params=pltpu.CompilerParams(
            dimension_semantics=("parallel", "parallel"),
        ),
    )(xin, codebooks, W_in, W_out)
    codes = codes_bqt.transpose(1, 0, 2)
    total_sq = jnp.sum(loss_parts) / 1024.0
    loss = total_sq * (1.25 / (b * t * _CODE_DIM))
    return z_q, loss, codes
